# Initial kernel scaffold; baseline (speedup 1.0000x reference)
import jax
import jax.numpy as jnp
from jax.experimental import pallas as pl


def kernel(outs_ct, bat_id, pit_id, fld_team_id, base1_run_id, base2_run_id, base3_run_id, away_score_ct, home_score_ct, inn_ct, bat_home_id, away_bat_lineup, home_bat_lineup, away_start_bat_ids, home_start_bat_ids, away_pit_id, home_pit_id, away_team_id, home_team_id, bat_table, pit_table, team_table, W1, b1, W2, b2, Wbd, bbd, Wr1, br1, Wr2, br2, Wr3, br3):
    raise NotImplementedError("write your pallas kernel here")


# speedup vs baseline: 6.0034x; 6.0034x over previous
import functools

import jax
import jax.numpy as jnp
from jax import lax
from jax.experimental import pallas as pl
from jax.experimental.pallas import tpu as pltpu
from jax.experimental.pallas import tpu_sc as plsc

B = 16384
EMB = 32
NW = 32
CHUNK = 128
NCHUNK = B // CHUNK
CPW = NCHUNK // NW
NB = 22
NP = 3
NT = 3


def _sc_gather_body(bat_t, pit_t, team_t, idxb_h, idxp_h, idxt_h,
                    xb_h, xp_h, xt_h,
                    idxb_v, idxp_v, idxt_v, bd, pd, td, sem):
    wid = lax.axis_index("s") * 2 + lax.axis_index("c")

    def chunk_body(c_local, carry):
        g = wid * CPW + c_local
        pltpu.sync_copy(idxb_h.at[g], idxb_v)
        pltpu.sync_copy(idxp_h.at[g], idxp_v)
        pltpu.sync_copy(idxt_h.at[g], idxt_v)

        def fire_bat(j, carry2):
            pltpu.make_async_copy(
                bat_t.at[idxb_v.at[j]],
                bd.at[pl.ds(j * CHUNK, CHUNK)], sem).start()
            return carry2

        lax.fori_loop(0, NB, fire_bat, 0)
        for j in range(NP):
            pltpu.make_async_copy(
                pit_t.at[idxp_v.at[j]],
                pd.at[pl.ds(j * CHUNK, CHUNK)], sem).start()
        for j in range(NT):
            pltpu.make_async_copy(
                team_t.at[idxt_v.at[j]],
                td.at[pl.ds(j * CHUNK, CHUNK)], sem).start()

        pltpu.make_async_copy(xb_h.at[g], bd, sem).wait()
        pltpu.make_async_copy(xp_h.at[g], pd, sem).wait()
        pltpu.make_async_copy(xt_h.at[g], td, sem).wait()

        pltpu.sync_copy(bd, xb_h.at[g])
        pltpu.sync_copy(pd, xp_h.at[g])
        pltpu.sync_copy(td, xt_h.at[g])
        return carry

    lax.fori_loop(0, CPW, chunk_body, 0)


_sc_gather = pl.kernel(
    _sc_gather_body,
    out_type=(
        jax.ShapeDtypeStruct((NCHUNK, CHUNK * NB, EMB), jnp.float32),
        jax.ShapeDtypeStruct((NCHUNK, CHUNK * NP, EMB), jnp.float32),
        jax.ShapeDtypeStruct((NCHUNK, CHUNK * NT, EMB), jnp.float32),
    ),
    mesh=plsc.VectorSubcoreMesh(
        core_axis_name="c", subcore_axis_name="s",
        num_cores=2, num_subcores=16),
    scratch_types=[
        pltpu.VMEM((NB, CHUNK), jnp.int32),
        pltpu.VMEM((NP, CHUNK), jnp.int32),
        pltpu.VMEM((NT, CHUNK), jnp.int32),
        pltpu.VMEM((CHUNK * NB, EMB), jnp.float32),
        pltpu.VMEM((CHUNK * NP, EMB), jnp.float32),
        pltpu.VMEM((CHUNK * NT, EMB), jnp.float32),
        pltpu.SemaphoreType.DMA,
    ],
    compiler_params=pltpu.CompilerParams(use_tc_tiling_on_sc=False),
)


def _mlp_body(xb, xp, xt, sc, w1b, w1p, w1t, w1s, b1, w2, b2, wh, bh,
              o0, o1, o2, o3):
    h1 = (jnp.dot(xb[...], w1b[...], preferred_element_type=jnp.float32)
          + jnp.dot(xp[...], w1p[...], preferred_element_type=jnp.float32)
          + jnp.dot(xt[...], w1t[...], preferred_element_type=jnp.float32)
          + jnp.dot(sc[...], w1s[...], preferred_element_type=jnp.float32)
          + b1[...])
    h1 = jnp.maximum(h1, 0.0)
    h2 = jnp.maximum(
        jnp.dot(h1, w2[...], preferred_element_type=jnp.float32) + b2[...], 0.0)
    lg = jnp.dot(h2, wh[...], preferred_element_type=jnp.float32) + bh[...]
    for i, o in enumerate((o0, o1, o2, o3)):
        sl = lg[:, i * 5:(i + 1) * 5]
        m = jnp.max(sl, axis=1, keepdims=True)
        e = jnp.exp(sl - m)
        o[...] = e / jnp.sum(e, axis=1, keepdims=True)


def _mlp_call(BB, xbf, xpf, xtf, scal, W1b, W1p, W1t, W1s, b1r, W2, b2r,
              Wh, bhm):
    nblk = B // BB
    full = lambda shape: pl.BlockSpec(shape, lambda i: (0, 0))
    return pl.pallas_call(
        _mlp_body,
        grid=(nblk,),
        in_specs=[
            pl.BlockSpec((BB, NB * EMB), lambda i: (i, 0)),
            pl.BlockSpec((BB, NP * EMB), lambda i: (i, 0)),
            pl.BlockSpec((BB, NT * EMB), lambda i: (i, 0)),
            pl.BlockSpec((BB, 8), lambda i: (i, 0)),
            full((NB * EMB, 512)),
            full((NP * EMB, 512)),
            full((NT * EMB, 512)),
            full((8, 512)),
            full((1, 512)),
            full((512, 256)),
            full((1, 256)),
            full((256, 20)),
            full((1, 20)),
        ],
        out_specs=[pl.BlockSpec((BB, 5), lambda i: (i, 0))] * 4,
        out_shape=[jax.ShapeDtypeStruct((B, 5), jnp.float32)] * 4,
    )(xbf, xpf, xtf, scal, W1b, W1p, W1t, W1s, b1r, W2, b2r, Wh, bhm)


def kernel(outs_ct, bat_id, pit_id, fld_team_id, base1_run_id, base2_run_id,
           base3_run_id, away_score_ct, home_score_ct, inn_ct, bat_home_id,
           away_bat_lineup, home_bat_lineup, away_start_bat_ids,
           home_start_bat_ids, away_pit_id, home_pit_id, away_team_id,
           home_team_id, bat_table, pit_table, team_table, W1, b1, W2, b2,
           Wbd, bbd, Wr1, br1, Wr2, br2, Wr3, br3):
    i32 = jnp.int32
    idxb = jnp.concatenate(
        [bat_id[:, None], base1_run_id[:, None], base2_run_id[:, None],
         base3_run_id[:, None], away_start_bat_ids, home_start_bat_ids],
        axis=1).astype(i32).reshape(NCHUNK, NB, CHUNK)
    idxp = jnp.stack([pit_id, away_pit_id, home_pit_id],
                     axis=1).astype(i32).reshape(NCHUNK, NP, CHUNK)
    idxt = jnp.stack([fld_team_id, away_team_id, home_team_id],
                     axis=1).astype(i32).reshape(NCHUNK, NT, CHUNK)

    xb, xp, xt = _sc_gather(bat_table, pit_table, team_table,
                            idxb, idxp, idxt)
    xbf = xb.reshape(B, NB * EMB)
    xpf = xp.reshape(B, NP * EMB)
    xtf = xt.reshape(B, NT * EMB)

    scal = jnp.concatenate(
        [outs_ct, away_score_ct, home_score_ct, inn_ct, bat_home_id,
         away_bat_lineup, home_bat_lineup,
         jnp.zeros((B, 1), jnp.float32)], axis=1)

    W1b = jnp.concatenate([W1[1:33], W1[97:193], W1[199:775]], axis=0)
    W1p = jnp.concatenate([W1[33:65], W1[775:839]], axis=0)
    W1t = jnp.concatenate([W1[65:97], W1[839:903]], axis=0)
    W1s = jnp.concatenate([W1[0:1], W1[193:199],
                           jnp.zeros((1, 512), jnp.float32)], axis=0)
    Wh = jnp.concatenate([Wbd, Wr1, Wr2, Wr3], axis=1)
    mask = jnp.array([0.0] * 5 + [0.0] * 5 + [0.0, -999.0, 0.0, 0.0, 0.0]
                     + [0.0, -999.0, -999.0, 0.0, 0.0], jnp.float32)
    bhm = (jnp.concatenate([bbd, br1, br2, br3]) + mask).reshape(1, 20)

    o0, o1, o2, o3 = _mlp_call(
        512, xbf, xpf, xtf, scal, W1b, W1p, W1t, W1s,
        b1.reshape(1, 512), W2, b2.reshape(1, 256), Wh, bhm)
    return (o0, o1, o2, o3)


# speedup vs baseline: 7.9465x; 1.3237x over previous
import jax
import jax.numpy as jnp
from jax import lax
from jax.experimental import pallas as pl
from jax.experimental.pallas import tpu as pltpu
from jax.experimental.pallas import tpu_sc as plsc

B = 16384
EMB = 32
NW = 32
CHUNK = 128
NCHUNK = B // CHUNK
CPW = NCHUNK // NW
NSEG = 28
GBYTES = NSEG * CHUNK * EMB * 4


def _sc_gather_body(bat_t, pit_t, team_t,
                    bat_id, base1, base2, base3, away_sb, home_sb,
                    pit_id, away_pit, home_pit,
                    fld_team, away_team, home_team,
                    x_h, idxbuf, gbuf, semi, semg, semw):
    wid = lax.axis_index("s") * 2 + lax.axis_index("c")

    singles = [bat_id, base1, base2, base3]

    def chunk_body(c_local, carry):
        r0 = (wid * CPW + c_local) * CHUNK
        rows = pl.ds(r0, CHUNK)

        def idx_dst(s):
            return idxbuf.at[pl.ds(s * CHUNK, CHUNK)]

        for s in range(4):
            pltpu.make_async_copy(singles[s].at[rows], idx_dst(s), semi).start()
        for j in range(9):
            pltpu.make_async_copy(away_sb.at[j, rows], idx_dst(4 + j), semi).start()
            pltpu.make_async_copy(home_sb.at[j, rows], idx_dst(13 + j), semi).start()
        for s, arr in ((22, pit_id), (23, away_pit), (24, home_pit),
                       (25, fld_team), (26, away_team), (27, home_team)):
            pltpu.make_async_copy(arr.at[rows], idx_dst(s), semi).start()
        pltpu.make_async_copy(bat_id.at[pl.ds(0, NSEG * CHUNK)], idxbuf, semi).wait()

        def table(s):
            return bat_t if s < 22 else (pit_t if s < 25 else team_t)

        for s in range(NSEG):
            pltpu.make_async_copy(
                table(s).at[idx_dst(s)],
                gbuf.at[pl.ds(s * CHUNK, CHUNK)], semg).start()
        pltpu.make_async_copy(
            x_h.at[0, pl.ds(0, NSEG * CHUNK), pl.ds(0, EMB)], gbuf, semg).wait()

        for s in range(NSEG):
            pltpu.make_async_copy(
                gbuf.at[pl.ds(s * CHUNK, CHUNK)],
                x_h.at[s // 4, rows, pl.ds((s % 4) * EMB, EMB)], semw).start()
        pltpu.make_async_copy(
            x_h.at[0, pl.ds(0, NSEG * CHUNK), pl.ds(0, EMB)], gbuf, semw).wait()
        return carry

    lax.fori_loop(0, CPW, chunk_body, 0)


_sc_gather = pl.kernel(
    _sc_gather_body,
    out_type=jax.ShapeDtypeStruct((7, B, 128), jnp.float32),
    mesh=plsc.VectorSubcoreMesh(
        core_axis_name="c", subcore_axis_name="s",
        num_cores=2, num_subcores=16),
    scratch_types=[
        pltpu.VMEM((NSEG * CHUNK,), jnp.int32),
        pltpu.VMEM((NSEG * CHUNK, EMB), jnp.float32),
        pltpu.SemaphoreType.DMA,
        pltpu.SemaphoreType.DMA,
        pltpu.SemaphoreType.DMA,
    ],
    compiler_params=pltpu.CompilerParams(use_tc_tiling_on_sc=False),
)


def _mlp_body(x, sc, w1, w1s, b1, w2, b2, wh, bh, o0, o1, o2, o3):
    h1 = jnp.dot(sc[...], w1s[...], preferred_element_type=jnp.float32)
    for t in range(7):
        h1 = h1 + jnp.dot(x[t], w1[t], preferred_element_type=jnp.float32)
    h1 = jnp.maximum(h1 + b1[...], 0.0)
    h2 = jnp.maximum(
        jnp.dot(h1, w2[...], preferred_element_type=jnp.float32) + b2[...], 0.0)
    lg = jnp.dot(h2, wh[...], preferred_element_type=jnp.float32) + bh[...]
    for i, o in enumerate((o0, o1, o2, o3)):
        sl = lg[:, i * 5:(i + 1) * 5]
        m = jnp.max(sl, axis=1, keepdims=True)
        e = jnp.exp(sl - m)
        o[...] = e / jnp.sum(e, axis=1, keepdims=True)


def _mlp_call(BB, x, scal, W1p, W1s, b1r, W2, b2r, Wh, bhm):
    nblk = B // BB
    full = lambda shape: pl.BlockSpec(shape, lambda i: tuple(0 for _ in shape))
    return pl.pallas_call(
        _mlp_body,
        grid=(nblk,),
        in_specs=[
            pl.BlockSpec((7, BB, 128), lambda i: (0, i, 0)),
            pl.BlockSpec((BB, 8), lambda i: (i, 0)),
            full((7, 128, 512)),
            full((8, 512)),
            full((1, 512)),
            full((512, 256)),
            full((1, 256)),
            full((256, 20)),
            full((1, 20)),
        ],
        out_specs=[pl.BlockSpec((BB, 5), lambda i: (i, 0))] * 4,
        out_shape=[jax.ShapeDtypeStruct((B, 5), jnp.float32)] * 4,
    )(x, scal, W1p, W1s, b1r, W2, b2r, Wh, bhm)


def kernel(outs_ct, bat_id, pit_id, fld_team_id, base1_run_id, base2_run_id,
           base3_run_id, away_score_ct, home_score_ct, inn_ct, bat_home_id,
           away_bat_lineup, home_bat_lineup, away_start_bat_ids,
           home_start_bat_ids, away_pit_id, home_pit_id, away_team_id,
           home_team_id, bat_table, pit_table, team_table, W1, b1, W2, b2,
           Wbd, bbd, Wr1, br1, Wr2, br2, Wr3, br3):
    i32 = jnp.int32
    x = _sc_gather(bat_table, pit_table, team_table,
                   bat_id.astype(i32), base1_run_id.astype(i32),
                   base2_run_id.astype(i32), base3_run_id.astype(i32),
                   away_start_bat_ids.astype(i32).T, home_start_bat_ids.astype(i32).T,
                   pit_id.astype(i32), away_pit_id.astype(i32),
                   home_pit_id.astype(i32),
                   fld_team_id.astype(i32), away_team_id.astype(i32),
                   home_team_id.astype(i32))

    scal = jnp.concatenate(
        [outs_ct, away_score_ct, home_score_ct, inn_ct, bat_home_id,
         away_bat_lineup, home_bat_lineup,
         jnp.zeros((B, 1), jnp.float32)], axis=1)

    W1p = jnp.concatenate(
        [W1[1:33], W1[97:193], W1[199:775],
         W1[33:65], W1[775:839],
         W1[65:97], W1[839:903]],
        axis=0).reshape(7, 128, 512)
    W1s = jnp.concatenate([W1[0:1], W1[193:199],
                           jnp.zeros((1, 512), jnp.float32)], axis=0)
    Wh = jnp.concatenate([Wbd, Wr1, Wr2, Wr3], axis=1)
    mask = jnp.array([0.0] * 11 + [-999.0, 0.0, 0.0, 0.0]
                     + [0.0, -999.0, -999.0, 0.0, 0.0], jnp.float32)
    bhm = (jnp.concatenate([bbd, br1, br2, br3]) + mask).reshape(1, 20)

    o0, o1, o2, o3 = _mlp_call(
        512, x, scal, W1p, W1s,
        b1.reshape(1, 512), W2, b2.reshape(1, 256), Wh, bhm)
    return (o0, o1, o2, o3)


# speedup vs baseline: 11.5836x; 1.4577x over previous
import jax
import jax.numpy as jnp
from jax import lax
from jax.experimental import pallas as pl
from jax.experimental.pallas import tpu as pltpu
from jax.experimental.pallas import tpu_sc as plsc

B = 16384
EMB = 32
NW = 32
CHUNK = 128
NCHUNK = B // CHUNK
CPW = NCHUNK // NW
NSEG = 28
GBYTES = NSEG * CHUNK * EMB * 4


def _sc_gather_body(bat_t, pit_t, team_t,
                    bat_id, base1, base2, base3, away_sb, home_sb,
                    pit_id, away_pit, home_pit,
                    fld_team, away_team, home_team,
                    x_h, idxbuf, gbuf, semi, semg, semw):
    wid = lax.axis_index("s") * 2 + lax.axis_index("c")

    singles = [bat_id, base1, base2, base3]

    def chunk_body(c_local, carry):
        r0 = (wid * CPW + c_local) * CHUNK
        rows = pl.ds(r0, CHUNK)

        def idx_dst(s):
            return idxbuf.at[pl.ds(s * CHUNK, CHUNK)]

        for s in range(4):
            pltpu.make_async_copy(singles[s].at[rows], idx_dst(s), semi).start()
        for j in range(9):
            pltpu.make_async_copy(away_sb.at[j, rows], idx_dst(4 + j), semi).start()
            pltpu.make_async_copy(home_sb.at[j, rows], idx_dst(13 + j), semi).start()
        for s, arr in ((22, pit_id), (23, away_pit), (24, home_pit),
                       (25, fld_team), (26, away_team), (27, home_team)):
            pltpu.make_async_copy(arr.at[rows], idx_dst(s), semi).start()
        pltpu.make_async_copy(bat_id.at[pl.ds(0, NSEG * CHUNK)], idxbuf, semi).wait()

        def table(s):
            return bat_t if s < 22 else (pit_t if s < 25 else team_t)

        for s in range(NSEG):
            pltpu.make_async_copy(
                table(s).at[idx_dst(s)],
                gbuf.at[pl.ds(s * CHUNK, CHUNK)], semg).start()
        pltpu.make_async_copy(
            x_h.at[0, pl.ds(0, NSEG * CHUNK), pl.ds(0, EMB)], gbuf, semg).wait()

        for s in range(NSEG):
            pltpu.make_async_copy(
                gbuf.at[pl.ds(s * CHUNK, CHUNK)],
                x_h.at[s // 4, rows, pl.ds((s % 4) * EMB, EMB)], semw).start()
        pltpu.make_async_copy(
            x_h.at[0, pl.ds(0, NSEG * CHUNK), pl.ds(0, EMB)], gbuf, semw).wait()
        return carry

    lax.fori_loop(0, CPW, chunk_body, 0)


_sc_gather = pl.kernel(
    _sc_gather_body,
    out_type=jax.ShapeDtypeStruct((7, B, 128), jnp.float32),
    mesh=plsc.VectorSubcoreMesh(
        core_axis_name="c", subcore_axis_name="s",
        num_cores=2, num_subcores=16),
    scratch_types=[
        pltpu.VMEM((NSEG * CHUNK,), jnp.int32),
        pltpu.VMEM((NSEG * CHUNK, EMB), jnp.float32),
        pltpu.SemaphoreType.DMA,
        pltpu.SemaphoreType.DMA,
        pltpu.SemaphoreType.DMA,
    ],
    compiler_params=pltpu.CompilerParams(use_tc_tiling_on_sc=False),
)


def _mlp_body(x, sc, w1, w1s, b1, w2, b2, wh, bh, o0, o1, o2, o3):
    bf16 = jnp.bfloat16
    xb = jnp.concatenate([x[t] for t in range(7)], axis=1).astype(bf16)
    h1 = jnp.dot(xb, w1[...], preferred_element_type=jnp.float32)
    h1 = h1 + jnp.dot(sc[...].T.astype(bf16), w1s[...],
                      preferred_element_type=jnp.float32)
    h1 = jnp.maximum(h1 + b1[...], 0.0).astype(bf16)
    h2 = jnp.maximum(
        jnp.dot(h1, w2[...], preferred_element_type=jnp.float32) + b2[...],
        0.0).astype(bf16)
    lg = jnp.dot(h2, wh[...], preferred_element_type=jnp.float32) + bh[...]
    lgt = lg.T
    for i, o in enumerate((o0, o1, o2, o3)):
        sl = lgt[i * 5:(i + 1) * 5, :]
        m = jnp.max(sl, axis=0, keepdims=True)
        e = jnp.exp(sl - m)
        o[...] = e / jnp.sum(e, axis=0, keepdims=True)


def _mlp_call(BB, x, scal, W1p, W1s, b1r, W2, b2r, Wh, bhm):
    nblk = B // BB
    full = lambda shape: pl.BlockSpec(shape, lambda i: tuple(0 for _ in shape))
    return pl.pallas_call(
        _mlp_body,
        grid=(nblk,),
        in_specs=[
            pl.BlockSpec((7, BB, 128), lambda i: (0, i, 0)),
            pl.BlockSpec((8, BB), lambda i: (0, i)),
            full((896, 512)),
            full((8, 512)),
            full((1, 512)),
            full((512, 256)),
            full((1, 256)),
            full((256, 20)),
            full((1, 20)),
        ],
        out_specs=[pl.BlockSpec((5, BB), lambda i: (0, i))] * 4,
        out_shape=[jax.ShapeDtypeStruct((5, B), jnp.float32)] * 4,
    )(x, scal, W1p, W1s, b1r, W2, b2r, Wh, bhm)


def kernel(outs_ct, bat_id, pit_id, fld_team_id, base1_run_id, base2_run_id,
           base3_run_id, away_score_ct, home_score_ct, inn_ct, bat_home_id,
           away_bat_lineup, home_bat_lineup, away_start_bat_ids,
           home_start_bat_ids, away_pit_id, home_pit_id, away_team_id,
           home_team_id, bat_table, pit_table, team_table, W1, b1, W2, b2,
           Wbd, bbd, Wr1, br1, Wr2, br2, Wr3, br3):
    i32 = jnp.int32
    x = _sc_gather(bat_table, pit_table, team_table,
                   bat_id.astype(i32), base1_run_id.astype(i32),
                   base2_run_id.astype(i32), base3_run_id.astype(i32),
                   away_start_bat_ids.astype(i32).T, home_start_bat_ids.astype(i32).T,
                   pit_id.astype(i32), away_pit_id.astype(i32),
                   home_pit_id.astype(i32),
                   fld_team_id.astype(i32), away_team_id.astype(i32),
                   home_team_id.astype(i32))

    scal = jnp.concatenate(
        [outs_ct.T, away_score_ct.T, home_score_ct.T, inn_ct.T, bat_home_id.T,
         away_bat_lineup.T, home_bat_lineup.T,
         jnp.zeros((1, B), jnp.float32)], axis=0)

    bf16 = jnp.bfloat16
    W1p = jnp.concatenate(
        [W1[1:33], W1[97:193], W1[199:775],
         W1[33:65], W1[775:839],
         W1[65:97], W1[839:903]],
        axis=0).astype(bf16)
    W1s = jnp.concatenate([W1[0:1], W1[193:199],
                           jnp.zeros((1, 512), jnp.float32)],
                          axis=0).astype(bf16)
    Wh = jnp.concatenate([Wbd, Wr1, Wr2, Wr3], axis=1).astype(bf16)
    mask = jnp.array([0.0] * 11 + [-999.0, 0.0, 0.0, 0.0]
                     + [0.0, -999.0, -999.0, 0.0, 0.0], jnp.float32)
    bhm = (jnp.concatenate([bbd, br1, br2, br3]) + mask).reshape(1, 20)

    o0, o1, o2, o3 = _mlp_call(
        512, x, scal, W1p, W1s,
        b1.reshape(1, 512), W2.astype(bf16), b2.reshape(1, 256), Wh, bhm)
    return (o0.T, o1.T, o2.T, o3.T)
